# baseline (speedup 1.0000x reference)
import jax
import jax.numpy as jnp
from jax import lax
from jax.experimental import pallas as pl
from jax.experimental.pallas import tpu as pltpu
from jax.experimental.pallas import tpu_sc as plsc

B, L, DIM = 1024, 200, 64
SEG = 3
NC, NS, LANES = 2, 16, 16
NW = NC * NS
N = B * L
PER_W = N // NW
CH = 128
G = PER_W // CH


def _body(xi_hbm, si_hbm, word_hbm, comb_hbm, out_hbm,
          idx_v, gidx_v, sidx_v, g_v, o_v, comb_vm, sem):
    c = lax.axis_index("c")
    s = lax.axis_index("s")
    wid = s * NC + c
    iota = lax.iota(jnp.int32, LANES)
    pltpu.sync_copy(comb_hbm, comb_vm)

    def chunk(g, carry):
        nbase = wid * PER_W + g * CH
        pltpu.sync_copy(xi_hbm.at[pl.ds(nbase, CH)], idx_v)
        pltpu.sync_copy(si_hbm.at[pl.ds(nbase, CH)], sidx_v)
        for i in range(CH // 16):
            sl = pl.ds(i * 16, 16)
            gidx_v[sl] = lax.shift_right_logical(idx_v[sl], 1)
        for k in range(CH // 128):
            ksl = pl.ds(k * 128, 128)
            pltpu.async_copy(word_hbm.at[gidx_v.at[ksl]], g_v.at[ksl],
                             sem).wait()
        for i in range(CH // 16):
            sl = pl.ds(i * 16, 16)
            rvec = iota + i * 16
            idxv = idx_v[sl]
            offv = (idxv & 1) * DIM
            cidxv = lax.rem(iota + (nbase + i * 16), L) * SEG + sidx_v[sl]

            def col(j, carry2):
                for u in range(4):
                    jj = j * 4 + u
                    jsp = jnp.full((LANES,), 0, jnp.int32) + jj
                    wv = plsc.load_gather(g_v, [rvec, offv + jj])
                    cv = plsc.load_gather(comb_vm, [cidxv, jsp])
                    plsc.store_scatter(o_v, [rvec, jsp], wv + cv)
                return carry2
            lax.fori_loop(0, DIM // 4, col, 0)
        pltpu.sync_copy(o_v, out_hbm.at[pl.ds(nbase, CH)])
        return carry

    lax.fori_loop(0, G, chunk, 0)


def kernel(x, segment_x, word_table, pos_emb, seg_table):
    xf = x.reshape(N).astype(jnp.int32)
    sf = segment_x.reshape(N).astype(jnp.int32)
    wt2 = word_table.reshape(500000, 2 * DIM)
    comb = (pos_emb[0, :L, :][:, None, :] + seg_table[None, :, :]
            ).reshape(L * SEG, DIM).astype(jnp.float32)
    mesh = plsc.VectorSubcoreMesh(core_axis_name="c", subcore_axis_name="s",
                                  num_cores=NC, num_subcores=NS)
    out = pl.kernel(
        _body,
        out_type=jax.ShapeDtypeStruct((N, DIM), jnp.float32),
        mesh=mesh,
        scratch_types=[
            pltpu.VMEM((CH,), jnp.int32),
            pltpu.VMEM((CH,), jnp.int32),
            pltpu.VMEM((CH,), jnp.int32),
            pltpu.VMEM((CH, 2 * DIM), jnp.float32),
            pltpu.VMEM((CH, DIM), jnp.float32),
            pltpu.VMEM((L * SEG, DIM), jnp.float32),
            pltpu.SemaphoreType.DMA,
        ],
        compiler_params=pltpu.CompilerParams(needs_layout_passes=False),
    )(xf, sf, wt2, comb)
    return out.reshape(B, L, DIM)


# speedup vs baseline: 1.8488x; 1.8488x over previous
import jax
import jax.numpy as jnp
from jax import lax
from jax.experimental import pallas as pl
from jax.experimental.pallas import tpu as pltpu
from jax.experimental.pallas import tpu_sc as plsc

B, L, DIM = 1024, 200, 64
SEG = 3
NC, NS, LANES = 2, 16, 16
NW = NC * NS
N = B * L
PER_W = N // NW
CH = 256
G = PER_W // CH
KB = CH // 128


def _body(xi_hbm, si_hbm, word_hbm, comb_hbm, out_hbm,
          idx_v, sidx_v, cidx_v, rows_v, crows_v,
          si0, si1, si2, sg0, sg1, so0, so1):
    c = lax.axis_index("c")
    s = lax.axis_index("s")
    wid = s * NC + c
    iota = lax.iota(jnp.int32, LANES)
    sem_i = (si0, si1, si2)
    sem_g = (sg0, sg1)
    sem_o = (so0, so1)

    def stage1(g):
        b = g % 3
        nbase = wid * PER_W + g * CH
        return [
            pltpu.async_copy(xi_hbm.at[pl.ds(nbase, CH)], idx_v.at[b],
                             sem_i[b]),
            pltpu.async_copy(si_hbm.at[pl.ds(nbase, CH)], sidx_v.at[b],
                             sem_i[b]),
        ]

    def stage2(g, idescs):
        bi = g % 3
        b = g % 2
        nbase = wid * PER_W + g * CH
        for d in idescs:
            d.wait()

        def cix(j, carry):
            sl = pl.ds(j * 16, 16)
            cidx_v[b, sl] = (lax.rem(iota + (nbase + j * 16), L) * SEG
                             + sidx_v[bi, sl])
            return carry
        lax.fori_loop(0, CH // 16, cix, 0)
        descs = []
        for k in range(KB):
            ksl = pl.ds(k * 128, 128)
            descs.append(pltpu.async_copy(
                word_hbm.at[idx_v.at[bi, ksl]], rows_v.at[b, ksl], sem_g[b]))
            descs.append(pltpu.async_copy(
                comb_hbm.at[cidx_v.at[b, ksl]], crows_v.at[b, ksl], sem_g[b]))
        return descs

    def stage3(g, gdescs):
        b = g % 2
        nbase = wid * PER_W + g * CH
        for d in gdescs:
            d.wait()

        def add(r, carry):
            for cc in range(DIM // 16):
                sl = pl.ds(cc * 16, 16)
                rows_v[b, r, sl] = rows_v[b, r, sl] + crows_v[b, r, sl]
            return carry
        lax.fori_loop(0, CH, add, 0)
        return pltpu.async_copy(rows_v.at[b], out_hbm.at[pl.ds(nbase, CH)],
                                sem_o[b])

    descs_i = {0: stage1(0), 1: stage1(1)}
    descs_g = {0: stage2(0, descs_i[0])}
    descs_o = {}
    for g in range(G):
        if g + 2 < G:
            descs_i[g + 2] = stage1(g + 2)
        if g + 1 < G:
            if g - 1 >= 0:
                descs_o[g - 1].wait()
            descs_g[g + 1] = stage2(g + 1, descs_i[g + 1])
        descs_o[g] = stage3(g, descs_g[g])
    descs_o[G - 2].wait()
    descs_o[G - 1].wait()


def kernel(x, segment_x, word_table, pos_emb, seg_table):
    xf = x.reshape(N).astype(jnp.int32)
    sf = segment_x.reshape(N).astype(jnp.int32)
    comb = (pos_emb[0, :L, :][:, None, :] + seg_table[None, :, :]
            ).reshape(L * SEG, DIM).astype(jnp.float32)
    mesh = plsc.VectorSubcoreMesh(core_axis_name="c", subcore_axis_name="s",
                                  num_cores=NC, num_subcores=NS)
    out = pl.kernel(
        _body,
        out_type=jax.ShapeDtypeStruct((N, DIM), jnp.float32),
        mesh=mesh,
        scratch_types=[
            pltpu.VMEM((3, CH), jnp.int32),
            pltpu.VMEM((3, CH), jnp.int32),
            pltpu.VMEM((2, CH), jnp.int32),
            pltpu.VMEM((2, CH, DIM), jnp.float32),
            pltpu.VMEM((2, CH, DIM), jnp.float32),
            pltpu.SemaphoreType.DMA,
            pltpu.SemaphoreType.DMA,
            pltpu.SemaphoreType.DMA,
            pltpu.SemaphoreType.DMA,
            pltpu.SemaphoreType.DMA,
            pltpu.SemaphoreType.DMA,
            pltpu.SemaphoreType.DMA,
        ],
        compiler_params=pltpu.CompilerParams(use_tc_tiling_on_sc=False),
    )(xf, sf, word_table, comb)
    return out.reshape(B, L, DIM)


# speedup vs baseline: 1.8894x; 1.0220x over previous
import jax
import jax.numpy as jnp
from jax import lax
from jax.experimental import pallas as pl
from jax.experimental.pallas import tpu as pltpu
from jax.experimental.pallas import tpu_sc as plsc

B, L, DIM = 1024, 200, 64
SEG = 3
NC, NS, LANES = 2, 16, 16
NW = NC * NS
N = B * L
PER_W = N // NW
CH = 200
G = PER_W // CH
SUBS = (0, 128)


def _body(xi_hbm, si_hbm, word_hbm, comb_hbm, out_hbm,
          idx_v, sidx_v, cidx_v, rows_v, crows_v,
          si0, si1, si2, sg0, sg1, so0, so1):
    c = lax.axis_index("c")
    s = lax.axis_index("s")
    wid = s * NC + c
    iota = lax.iota(jnp.int32, LANES)
    sem_i = (si0, si1, si2)
    sem_g = (sg0, sg1)
    sem_o = (so0, so1)

    def stage1(g):
        b = g % 3
        nbase = wid * PER_W + g * CH
        return [
            pltpu.async_copy(xi_hbm.at[pl.ds(nbase, CH)], idx_v.at[b],
                             sem_i[b]),
            pltpu.async_copy(si_hbm.at[pl.ds(nbase, CH)], sidx_v.at[b],
                             sem_i[b]),
        ]

    def stage2(g, idescs):
        bi = g % 3
        b = g % 2
        nbase = wid * PER_W + g * CH
        for d in idescs:
            d.wait()

        for j in range(CH // 16):
            sl = pl.ds(j * 16, 16)
            cidx_v[b, sl] = (lax.rem(iota + (nbase + j * 16), L) * SEG
                             + sidx_v[bi, sl])
        sl = pl.ds(CH - 16, 16)
        cidx_v[b, sl] = (lax.rem(iota + (nbase + CH - 16), L) * SEG
                         + sidx_v[bi, sl])
        descs = []
        for k, st in enumerate(SUBS):
            w = min(128, CH - st)
            ksl = pl.ds(st, w)
            descs.append(pltpu.async_copy(
                word_hbm.at[idx_v.at[bi, ksl]], rows_v.at[b, ksl], sem_g[b]))
            descs.append(pltpu.async_copy(
                comb_hbm.at[cidx_v.at[b, ksl]], crows_v.at[b, ksl], sem_g[b]))
        return descs

    def stage3(g, gdescs):
        b = g % 2
        nbase = wid * PER_W + g * CH
        for d in gdescs:
            d.wait()

        def add(r, carry):
            for cc in range(DIM // 16):
                sl = pl.ds(cc * 16, 16)
                crows_v[b, r, sl] = rows_v[b, r, sl] + crows_v[b, r, sl]
            return carry
        lax.fori_loop(0, CH, add, 0)
        return pltpu.async_copy(crows_v.at[b], out_hbm.at[pl.ds(nbase, CH)],
                                sem_o[b])

    descs_i = {0: stage1(0), 1: stage1(1)}
    descs_g = {0: stage2(0, descs_i[0])}
    descs_o = {}
    for g in range(G):
        if g + 2 < G:
            descs_i[g + 2] = stage1(g + 2)
        if g + 1 < G:
            if g - 1 >= 0:
                descs_o[g - 1].wait()
            descs_g[g + 1] = stage2(g + 1, descs_i[g + 1])
        descs_o[g] = stage3(g, descs_g[g])
    descs_o[G - 2].wait()
    descs_o[G - 1].wait()


def kernel(x, segment_x, word_table, pos_emb, seg_table):
    xf = x.reshape(N).astype(jnp.int32)
    sf = segment_x.reshape(N).astype(jnp.int32)
    wt_pad = jnp.pad(word_table, ((0, 0), (0, 128 - DIM)))
    comb = (pos_emb[0, :L, :][:, None, :] + seg_table[None, :, :]
            ).reshape(L * SEG, DIM).astype(jnp.float32)
    mesh = plsc.VectorSubcoreMesh(core_axis_name="c", subcore_axis_name="s",
                                  num_cores=NC, num_subcores=NS)
    out = pl.kernel(
        _body,
        out_type=jax.ShapeDtypeStruct((N, DIM), jnp.float32),
        mesh=mesh,
        scratch_types=[
            pltpu.VMEM((3, CH), jnp.int32),
            pltpu.VMEM((3, CH), jnp.int32),
            pltpu.VMEM((2, CH), jnp.int32),
            pltpu.VMEM((2, CH, 2 * DIM), jnp.float32),
            pltpu.VMEM((2, CH, DIM), jnp.float32),
            pltpu.SemaphoreType.DMA,
            pltpu.SemaphoreType.DMA,
            pltpu.SemaphoreType.DMA,
            pltpu.SemaphoreType.DMA,
            pltpu.SemaphoreType.DMA,
            pltpu.SemaphoreType.DMA,
            pltpu.SemaphoreType.DMA,
        ],
        compiler_params=pltpu.CompilerParams(use_tc_tiling_on_sc=False),
    )(xf, sf, wt_pad, comb)
    return out.reshape(B, L, DIM)


# speedup vs baseline: 1.8916x; 1.0011x over previous
import jax
import jax.numpy as jnp
from jax import lax
from jax.experimental import pallas as pl
from jax.experimental.pallas import tpu as pltpu
from jax.experimental.pallas import tpu_sc as plsc

B, L, DIM = 1024, 200, 64
SEG = 3
NC, NS, LANES = 2, 16, 16
NW = NC * NS
N = B * L
PER_W = N // NW
CH = 200
G = PER_W // CH
SUBS = (0, 128)


V = 1000000


def _body(xi_hbm, si_hbm, word_hbm, comb_hbm, out_hbm,
          idx_v, sidx_v, cidx_v, rows_v, crows_v,
          si0, si1, si2, sg0, sg1, so0, so1):
    c = lax.axis_index("c")
    s = lax.axis_index("s")
    wid = s * NC + c
    iota = lax.iota(jnp.int32, LANES)
    sem_i = (si0, si1, si2)
    sem_g = (sg0, sg1)
    sem_o = (so0, so1)

    def stage1(g):
        b = g % 3
        nbase = wid * PER_W + g * CH
        return [
            pltpu.async_copy(xi_hbm.at[pl.ds(nbase, CH)], idx_v.at[b],
                             sem_i[b]),
            pltpu.async_copy(si_hbm.at[pl.ds(nbase, CH)], sidx_v.at[b],
                             sem_i[b]),
        ]

    def stage2(g, idescs):
        bi = g % 3
        b = g % 2
        nbase = wid * PER_W + g * CH
        for d in idescs:
            d.wait()

        for j in range(CH // 16):
            sl = pl.ds(j * 16, 16)
            cidx_v[b, sl] = (lax.rem(iota + (nbase + j * 16), L) * SEG
                             + sidx_v[bi, sl])
        sl = pl.ds(CH - 16, 16)
        cidx_v[b, sl] = (lax.rem(iota + (nbase + CH - 16), L) * SEG
                         + sidx_v[bi, sl])
        descs = []
        for k, st in enumerate(SUBS):
            w = min(128, CH - st)
            ksl = pl.ds(st, w)
            descs.append(pltpu.async_copy(
                word_hbm.at[idx_v.at[bi, ksl]], rows_v.at[b, ksl], sem_g[b]))
            descs.append(pltpu.async_copy(
                comb_hbm.at[cidx_v.at[b, ksl]], crows_v.at[b, ksl], sem_g[b]))
        return descs

    def stage3(g, gdescs):
        b = g % 2
        nbase = wid * PER_W + g * CH
        for d in gdescs:
            d.wait()

        def add(r, carry):
            for cc in range(DIM // 16):
                sl = pl.ds(cc * 16, 16)
                crows_v[b, r, sl] = rows_v[b, r, sl] + crows_v[b, r, sl]
            return carry
        lax.fori_loop(0, CH, add, 0)
        return pltpu.async_copy(crows_v.at[b], out_hbm.at[wid * G + g],
                                sem_o[b])

    descs_i = {0: stage1(0), 1: stage1(1)}
    descs_g = {0: stage2(0, descs_i[0])}
    descs_o = {}
    for g in range(G):
        if g + 2 < G:
            descs_i[g + 2] = stage1(g + 2)
        if g + 1 < G:
            if g - 1 >= 0:
                descs_o[g - 1].wait()
            descs_g[g + 1] = stage2(g + 1, descs_i[g + 1])
        descs_o[g] = stage3(g, descs_g[g])
    descs_o[G - 2].wait()
    descs_o[G - 1].wait()


def kernel(x, segment_x, word_table, pos_emb, seg_table):
    xf = x.reshape(N).astype(jnp.int32)
    sf = segment_x.reshape(N).astype(jnp.int32)
    comb = (pos_emb[0, :L, :][:, None, :] + seg_table[None, :, :]
            ).reshape(L * SEG, DIM).astype(jnp.float32)
    mesh = plsc.VectorSubcoreMesh(core_axis_name="c", subcore_axis_name="s",
                                  num_cores=NC, num_subcores=NS)
    wt_pad = jnp.concatenate(
        [word_table, jnp.zeros((V, 2 * DIM - DIM), jnp.float32)], axis=1)
    out = pl.kernel(
        _body,
        out_type=jax.ShapeDtypeStruct((B, L, DIM), jnp.float32),
        mesh=mesh,
        scratch_types=[
            pltpu.VMEM((3, CH), jnp.int32),
            pltpu.VMEM((3, CH), jnp.int32),
            pltpu.VMEM((2, CH), jnp.int32),
            pltpu.VMEM((2, CH, 2 * DIM), jnp.float32),
            pltpu.VMEM((2, CH, DIM), jnp.float32),
            pltpu.SemaphoreType.DMA,
            pltpu.SemaphoreType.DMA,
            pltpu.SemaphoreType.DMA,
            pltpu.SemaphoreType.DMA,
            pltpu.SemaphoreType.DMA,
            pltpu.SemaphoreType.DMA,
            pltpu.SemaphoreType.DMA,
        ],
        compiler_params=pltpu.CompilerParams(use_tc_tiling_on_sc=False),
    )(xf, sf, wt_pad, comb)
    return out
